# baseline (speedup 1.0000x reference)
import functools

import jax
import jax.numpy as jnp
from jax import lax
from jax.experimental import pallas as pl
from jax.experimental.pallas import tpu as pltpu
from jax.experimental.pallas import tpu_sc as plsc

N_SENT = 4096
SEQ = 50
SEQ_P = 56
N_ROWS = N_SENT * SEQ_P
D_EMB = 128
D_HID = 1024
N_WORKERS = 32
CHUNK = 112
N_CHUNKS = N_ROWS // (N_WORKERS * CHUNK)

@functools.cache
def _make_sc_gather():
    mesh = plsc.VectorSubcoreMesh(core_axis_name="c", subcore_axis_name="s")

    @functools.partial(
        pl.kernel,
        out_type=jax.ShapeDtypeStruct((N_ROWS, D_EMB), jnp.float32),
        mesh=mesh,
        scratch_types=[
            pltpu.VMEM((N_CHUNKS * CHUNK,), jnp.int32),
            pltpu.VMEM((2, CHUNK, D_EMB), jnp.float32),
            pltpu.SemaphoreType.DMA,
            pltpu.SemaphoreType.DMA,
        ],
    )
    def _sc_gather(ids_hbm, table_hbm, out_hbm, idx_v, rows_v, gsem, ssem):
        wid = lax.axis_index("s") * 2 + lax.axis_index("c")
        pltpu.sync_copy(ids_hbm.at[pl.ds(wid * N_CHUNKS * CHUNK, N_CHUNKS * CHUNK)], idx_v)
        base_row = wid * N_CHUNKS * CHUNK

        def chunk_pair(j2, carry):
            for b in range(2):
                j = j2 + b

                @pl.when(j < N_CHUNKS)
                def _():
                    pltpu.async_copy(
                        table_hbm.at[idx_v.at[pl.ds(j * CHUNK, CHUNK)]],
                        rows_v.at[b],
                        gsem,
                    ).wait()
                    pltpu.sync_copy(
                        rows_v.at[b],
                        out_hbm.at[pl.ds(base_row + j * CHUNK, CHUNK)],
                    )
            return carry

        lax.fori_loop(
            0, (N_CHUNKS + 1) // 2, lambda i, c: chunk_pair(i * 2, c), 0
        )

    return _sc_gather


SENT_BLK = 8
ROW_BLK = SENT_BLK * SEQ_P


def _proj_ln_body(emb_ref, w_ref, g_ref, b_ref, out_ref):
    x = emb_ref[...]
    p = jnp.dot(x, w_ref[...], preferred_element_type=jnp.float32)
    mu = jnp.mean(p, axis=-1, keepdims=True)
    var = jnp.mean((p - mu) ** 2, axis=-1, keepdims=True)
    inv = lax.rsqrt(var + 1e-12)
    y = (p - mu) * inv * g_ref[...] + b_ref[...]
    out_ref[...] = y.reshape(SENT_BLK, SEQ_P, D_HID)[:, :SEQ, :]


_proj_ln = pl.pallas_call(
    _proj_ln_body,
    grid=(N_ROWS // ROW_BLK,),
    in_specs=[
        pl.BlockSpec((ROW_BLK, D_EMB), lambda i: (i, 0)),
        pl.BlockSpec((D_EMB, D_HID), lambda i: (0, 0)),
        pl.BlockSpec((1, D_HID), lambda i: (0, 0)),
        pl.BlockSpec((1, D_HID), lambda i: (0, 0)),
    ],
    out_specs=pl.BlockSpec((SENT_BLK, SEQ, D_HID), lambda i: (i, 0, 0)),
    out_shape=jax.ShapeDtypeStruct((N_SENT, SEQ, D_HID), jnp.float32),
)


def kernel(entity_ids, table, W, gamma, beta):
    ids_pad = jnp.pad(entity_ids, ((0, 0), (0, SEQ_P - SEQ))).reshape(N_ROWS)
    rows = _make_sc_gather()(ids_pad, table)
    return _proj_ln(rows, W, gamma.reshape(1, D_HID), beta.reshape(1, D_HID))


# speedup vs baseline: 1.7372x; 1.7372x over previous
import functools

import jax
import jax.numpy as jnp
from jax import lax
from jax.experimental import pallas as pl
from jax.experimental.pallas import tpu as pltpu
from jax.experimental.pallas import tpu_sc as plsc

N_SENT = 4096
SEQ = 50
SEQ_P = 56
N_ROWS = N_SENT * SEQ_P
D_EMB = 128
D_HID = 1024
N_WORKERS = 32
CHUNK = 112
N_CHUNKS = N_ROWS // (N_WORKERS * CHUNK)

@functools.cache
def _make_sc_gather():
    mesh = plsc.VectorSubcoreMesh(core_axis_name="c", subcore_axis_name="s")

    @functools.partial(
        pl.kernel,
        out_type=jax.ShapeDtypeStruct((N_ROWS, D_EMB), jnp.float32),
        mesh=mesh,
        scratch_types=[
            pltpu.VMEM((N_CHUNKS * CHUNK,), jnp.int32),
            pltpu.VMEM((2, CHUNK, D_EMB), jnp.float32),
            pltpu.SemaphoreType.DMA,
            pltpu.SemaphoreType.DMA,
        ],
    )
    def _sc_gather(ids_hbm, table_hbm, out_hbm, idx_v, rows_v, gsem, ssem):
        wid = lax.axis_index("s") * 2 + lax.axis_index("c")
        pltpu.sync_copy(ids_hbm.at[pl.ds(wid * N_CHUNKS * CHUNK, N_CHUNKS * CHUNK)], idx_v)
        base_row = wid * N_CHUNKS * CHUNK

        def chunk_pair(j2, carry):
            for b in range(2):
                j = j2 + b

                @pl.when(j < N_CHUNKS)
                def _():
                    pltpu.async_copy(
                        table_hbm.at[idx_v.at[pl.ds(j * CHUNK, CHUNK)]],
                        rows_v.at[b],
                        gsem,
                    ).wait()
                    pltpu.sync_copy(
                        rows_v.at[b],
                        out_hbm.at[pl.ds(base_row + j * CHUNK, CHUNK)],
                    )
            return carry

        lax.fori_loop(
            0, (N_CHUNKS + 1) // 2, lambda i, c: chunk_pair(i * 2, c), 0
        )

    return _sc_gather


SENT_BLK = 8
ROW_BLK = SENT_BLK * SEQ_P


def _proj_ln_body(emb_ref, w_ref, g_ref, b_ref, out_ref):
    x = emb_ref[...]
    p = jnp.dot(x, w_ref[...], preferred_element_type=jnp.float32)
    mu = jnp.mean(p, axis=-1, keepdims=True)
    var = jnp.mean((p - mu) ** 2, axis=-1, keepdims=True)
    inv = lax.rsqrt(var + 1e-12)
    y = (p - mu) * inv * g_ref[...] + b_ref[...]
    out_ref[...] = y.reshape(SENT_BLK, SEQ_P, D_HID)[:, :SEQ, :]


_proj_ln = pl.pallas_call(
    _proj_ln_body,
    grid=(N_ROWS // ROW_BLK,),
    in_specs=[
        pl.BlockSpec((ROW_BLK, D_EMB), lambda i: (i, 0)),
        pl.BlockSpec((D_EMB, D_HID), lambda i: (0, 0)),
        pl.BlockSpec((1, D_HID), lambda i: (0, 0)),
        pl.BlockSpec((1, D_HID), lambda i: (0, 0)),
    ],
    out_specs=pl.BlockSpec((SENT_BLK, SEQ, D_HID), lambda i: (i, 0, 0)),
    out_shape=jax.ShapeDtypeStruct((N_SENT, SEQ, D_HID), jnp.float32),
)


def kernel(entity_ids, table, W, gamma, beta):
    ids_pad = jnp.concatenate(
        [entity_ids, entity_ids[:, : SEQ_P - SEQ]], axis=1
    ).reshape(N_ROWS)
    rows = _make_sc_gather()(ids_pad, table)
    return _proj_ln(rows, W, gamma.reshape(1, D_HID), beta.reshape(1, D_HID))


# speedup vs baseline: 2.0597x; 1.1856x over previous
import functools

import jax
import jax.numpy as jnp
from jax import lax
from jax.experimental import pallas as pl
from jax.experimental.pallas import tpu as pltpu
from jax.experimental.pallas import tpu_sc as plsc

N_SENT = 4096
SEQ = 50
SEQ_P = 56
N_ROWS = N_SENT * SEQ_P
D_EMB = 128
D_HID = 1024
N_WORKERS = 32
CHUNK = 112
N_CHUNKS = N_ROWS // (N_WORKERS * CHUNK)

@functools.cache
def _make_sc_gather():
    mesh = plsc.VectorSubcoreMesh(core_axis_name="c", subcore_axis_name="s")

    @functools.partial(
        pl.kernel,
        out_type=jax.ShapeDtypeStruct((N_ROWS, D_EMB), jnp.float32),
        mesh=mesh,
        scratch_types=[
            pltpu.VMEM((N_CHUNKS * CHUNK,), jnp.int32),
            pltpu.VMEM((2, CHUNK, D_EMB), jnp.float32),
            pltpu.SemaphoreType.DMA,
            pltpu.SemaphoreType.DMA,
        ],
    )
    def _sc_gather(ids_hbm, table_hbm, out_hbm, idx_v, rows_v, gsem, ssem):
        wid = lax.axis_index("s") * 2 + lax.axis_index("c")
        pltpu.sync_copy(ids_hbm.at[pl.ds(wid * N_CHUNKS * CHUNK, N_CHUNKS * CHUNK)], idx_v)
        base_row = wid * N_CHUNKS * CHUNK

        def chunk_pair(j2, carry):
            for b in range(2):
                j = j2 + b

                @pl.when(j < N_CHUNKS)
                def _():
                    pltpu.async_copy(
                        table_hbm.at[idx_v.at[pl.ds(j * CHUNK, CHUNK)]],
                        rows_v.at[b],
                        gsem,
                    ).wait()
                    pltpu.sync_copy(
                        rows_v.at[b],
                        out_hbm.at[pl.ds(base_row + j * CHUNK, CHUNK)],
                    )
            return carry

        lax.fori_loop(
            0, (N_CHUNKS + 1) // 2, lambda i, c: chunk_pair(i * 2, c), 0
        )

    return _sc_gather


SENT_BLK = 32
ROW_BLK = SENT_BLK * SEQ_P


def _proj_ln_body(emb_ref, w_ref, g_ref, b_ref, out_ref):
    x = emb_ref[...]
    p = jnp.dot(x, w_ref[...], preferred_element_type=jnp.float32)
    mu = jnp.mean(p, axis=-1, keepdims=True)
    var = jnp.mean((p - mu) ** 2, axis=-1, keepdims=True)
    inv = lax.rsqrt(var + 1e-12)
    y = (p - mu) * inv * g_ref[...] + b_ref[...]
    out_ref[...] = y.reshape(SENT_BLK, SEQ_P, D_HID)[:, :SEQ, :]


_proj_ln = pl.pallas_call(
    _proj_ln_body,
    grid=(N_ROWS // ROW_BLK,),
    in_specs=[
        pl.BlockSpec((ROW_BLK, D_EMB), lambda i: (i, 0)),
        pl.BlockSpec((D_EMB, D_HID), lambda i: (0, 0)),
        pl.BlockSpec((1, D_HID), lambda i: (0, 0)),
        pl.BlockSpec((1, D_HID), lambda i: (0, 0)),
    ],
    out_specs=pl.BlockSpec((SENT_BLK, SEQ, D_HID), lambda i: (i, 0, 0)),
    out_shape=jax.ShapeDtypeStruct((N_SENT, SEQ, D_HID), jnp.float32),
)


def kernel(entity_ids, table, W, gamma, beta):
    ids_pad = jnp.concatenate(
        [entity_ids, entity_ids[:, : SEQ_P - SEQ]], axis=1
    ).reshape(N_ROWS)
    rows = _make_sc_gather()(ids_pad, table)
    return _proj_ln(rows, W, gamma.reshape(1, D_HID), beta.reshape(1, D_HID))


# speedup vs baseline: 2.1193x; 1.0290x over previous
import functools

import jax
import jax.numpy as jnp
from jax import lax
from jax.experimental import pallas as pl
from jax.experimental.pallas import tpu as pltpu
from jax.experimental.pallas import tpu_sc as plsc

N_SENT = 4096
SEQ = 50
SEQ_P = 56
N_ROWS = N_SENT * SEQ_P
D_EMB = 128
D_HID = 1024
N_WORKERS = 32
CHUNK = 112
N_CHUNKS = N_ROWS // (N_WORKERS * CHUNK)

@functools.cache
def _make_sc_gather():
    mesh = plsc.VectorSubcoreMesh(core_axis_name="c", subcore_axis_name="s")

    @functools.partial(
        pl.kernel,
        out_type=jax.ShapeDtypeStruct((N_ROWS, D_EMB), jnp.float32),
        mesh=mesh,
        scratch_types=[
            pltpu.VMEM((N_CHUNKS * CHUNK,), jnp.int32),
            pltpu.VMEM((2, CHUNK, D_EMB), jnp.float32),
            pltpu.SemaphoreType.DMA,
            pltpu.SemaphoreType.DMA,
        ],
    )
    def _sc_gather(ids_hbm, table_hbm, out_hbm, idx_v, rows_v, gsem, ssem):
        wid = lax.axis_index("s") * 2 + lax.axis_index("c")
        pltpu.sync_copy(ids_hbm.at[pl.ds(wid * N_CHUNKS * CHUNK, N_CHUNKS * CHUNK)], idx_v)
        base_row = wid * N_CHUNKS * CHUNK

        def chunk_pair(j2, carry):
            for b in range(2):
                j = j2 + b

                @pl.when(j < N_CHUNKS)
                def _():
                    pltpu.async_copy(
                        table_hbm.at[idx_v.at[pl.ds(j * CHUNK, CHUNK)]],
                        rows_v.at[b],
                        gsem,
                    ).wait()
                    pltpu.sync_copy(
                        rows_v.at[b],
                        out_hbm.at[pl.ds(base_row + j * CHUNK, CHUNK)],
                    )
            return carry

        lax.fori_loop(
            0, (N_CHUNKS + 1) // 2, lambda i, c: chunk_pair(i * 2, c), 0
        )

    return _sc_gather


SENT_BLK = 64
ROW_BLK = SENT_BLK * SEQ_P


def _proj_ln_body(emb_ref, w_ref, g_ref, b_ref, out_ref):
    x = emb_ref[...]
    p = jnp.dot(x, w_ref[...], preferred_element_type=jnp.float32)
    mu = jnp.mean(p, axis=-1, keepdims=True)
    var = jnp.mean((p - mu) ** 2, axis=-1, keepdims=True)
    inv = lax.rsqrt(var + 1e-12)
    y = (p - mu) * inv * g_ref[...] + b_ref[...]
    out_ref[...] = y.reshape(SENT_BLK, SEQ_P, D_HID)[:, :SEQ, :]


_proj_ln = pl.pallas_call(
    _proj_ln_body,
    grid=(N_ROWS // ROW_BLK,),
    in_specs=[
        pl.BlockSpec((ROW_BLK, D_EMB), lambda i: (i, 0)),
        pl.BlockSpec((D_EMB, D_HID), lambda i: (0, 0)),
        pl.BlockSpec((1, D_HID), lambda i: (0, 0)),
        pl.BlockSpec((1, D_HID), lambda i: (0, 0)),
    ],
    out_specs=pl.BlockSpec((SENT_BLK, SEQ, D_HID), lambda i: (i, 0, 0)),
    out_shape=jax.ShapeDtypeStruct((N_SENT, SEQ, D_HID), jnp.float32),
)


def kernel(entity_ids, table, W, gamma, beta):
    ids_pad = jnp.concatenate(
        [entity_ids, entity_ids[:, : SEQ_P - SEQ]], axis=1
    ).reshape(N_ROWS)
    rows = _make_sc_gather()(ids_pad, table)
    return _proj_ln(rows, W, gamma.reshape(1, D_HID), beta.reshape(1, D_HID))


# speedup vs baseline: 2.1511x; 1.0150x over previous
import functools

import jax
import jax.numpy as jnp
from jax import lax
from jax.experimental import pallas as pl
from jax.experimental.pallas import tpu as pltpu
from jax.experimental.pallas import tpu_sc as plsc

N_SENT = 4096
SEQ = 50
SEQ_P = 56
N_ROWS = N_SENT * SEQ_P
D_EMB = 128
D_HID = 1024
N_WORKERS = 32
CHUNK = 112
N_CHUNKS = N_ROWS // (N_WORKERS * CHUNK)

@functools.cache
def _make_sc_gather():
    mesh = plsc.VectorSubcoreMesh(core_axis_name="c", subcore_axis_name="s")

    @functools.partial(
        pl.kernel,
        out_type=jax.ShapeDtypeStruct((N_ROWS, D_EMB), jnp.float32),
        mesh=mesh,
        scratch_types=[
            pltpu.VMEM((N_CHUNKS * CHUNK,), jnp.int32),
            pltpu.VMEM((2, CHUNK, D_EMB), jnp.float32),
            pltpu.SemaphoreType.DMA,
            pltpu.SemaphoreType.DMA,
            pltpu.SemaphoreType.DMA,
            pltpu.SemaphoreType.DMA,
        ],
    )
    def _sc_gather(ids_hbm, table_hbm, out_hbm, idx_v, rows_v, g0, g1, s0, s1):
        wid = lax.axis_index("s") * 2 + lax.axis_index("c")
        pltpu.sync_copy(
            ids_hbm.at[pl.ds(wid * N_CHUNKS * CHUNK, N_CHUNKS * CHUNK)], idx_v
        )
        base_row = wid * N_CHUNKS * CHUNK
        gsem = (g0, g1)
        ssem = (s0, s1)

        def start_gather(j, b):
            pltpu.async_copy(
                table_hbm.at[idx_v.at[pl.ds(j * CHUNK, CHUNK)]],
                rows_v.at[b],
                gsem[b],
            )

        def start_scatter(j, b):
            pltpu.async_copy(
                rows_v.at[b],
                out_hbm.at[pl.ds(base_row + j * CHUNK, CHUNK)],
                ssem[b],
            )

        def wait_gather(b):
            pltpu.make_async_copy(
                table_hbm.at[pl.ds(0, CHUNK)], rows_v.at[b], gsem[b]
            ).wait()

        def wait_scatter(b):
            pltpu.make_async_copy(
                rows_v.at[b], out_hbm.at[pl.ds(0, CHUNK)], ssem[b]
            ).wait()

        start_gather(0, 0)

        def chunk_pair(j2, carry):
            for b in range(2):
                j = j2 + b
                wait_gather(b)
                start_scatter(j, b)

                @pl.when(j > 0)
                def _():
                    wait_scatter(1 - b)

                @pl.when(j + 1 < N_CHUNKS)
                def _():
                    start_gather(j + 1, 1 - b)
            return carry

        lax.fori_loop(0, N_CHUNKS // 2, lambda i, c: chunk_pair(i * 2, c), 0)
        wait_scatter((N_CHUNKS - 1) % 2)

    return _sc_gather


SENT_BLK = 64
ROW_BLK = SENT_BLK * SEQ_P


def _proj_ln_body(emb_ref, w_ref, g_ref, b_ref, out_ref):
    x = emb_ref[...]
    p = jnp.dot(x, w_ref[...], preferred_element_type=jnp.float32)
    mu = jnp.mean(p, axis=-1, keepdims=True)
    var = jnp.mean((p - mu) ** 2, axis=-1, keepdims=True)
    inv = lax.rsqrt(var + 1e-12)
    y = (p - mu) * inv * g_ref[...] + b_ref[...]
    out_ref[...] = y.reshape(SENT_BLK, SEQ_P, D_HID)[:, :SEQ, :]


_proj_ln = pl.pallas_call(
    _proj_ln_body,
    grid=(N_ROWS // ROW_BLK,),
    in_specs=[
        pl.BlockSpec((ROW_BLK, D_EMB), lambda i: (i, 0)),
        pl.BlockSpec((D_EMB, D_HID), lambda i: (0, 0)),
        pl.BlockSpec((1, D_HID), lambda i: (0, 0)),
        pl.BlockSpec((1, D_HID), lambda i: (0, 0)),
    ],
    out_specs=pl.BlockSpec((SENT_BLK, SEQ, D_HID), lambda i: (i, 0, 0)),
    out_shape=jax.ShapeDtypeStruct((N_SENT, SEQ, D_HID), jnp.float32),
)


def kernel(entity_ids, table, W, gamma, beta):
    ids_pad = jnp.concatenate(
        [entity_ids, entity_ids[:, : SEQ_P - SEQ]], axis=1
    ).reshape(N_ROWS)
    rows = _make_sc_gather()(ids_pad, table)
    return _proj_ln(rows, W, gamma.reshape(1, D_HID), beta.reshape(1, D_HID))
